# baseline (speedup 1.0000x reference)
import jax
import jax.numpy as jnp
from jax.experimental import pallas as pl
from jax.experimental.pallas import tpu as pltpu

_HT = 128


def _pool_body(x_ref, m_ref, unch_ref, ch_ref, sums_ref, cnt_ref):
    b = pl.program_id(0)
    h = pl.program_id(1)

    @pl.when((b == 0) & (h == 0))
    def _init():
        sums_ref[...] = jnp.zeros_like(sums_ref)
        cnt_ref[0, 0] = jnp.float32(0.0)

    xb = x_ref[0]
    mb = m_ref[0].astype(jnp.float32)
    s1 = jnp.sum(xb * mb[None, :, :], axis=(1, 2))
    s0 = jnp.sum(xb, axis=(1, 2))
    sums_ref[...] += jnp.stack([s1, s0])
    cnt_ref[0, 0] += jnp.sum(mb)

    @pl.when((b == pl.num_programs(0) - 1) & (h == pl.num_programs(1) - 1))
    def _finish():
        n_ch = cnt_ref[0, 0]
        n_tot = jnp.float32(m_ref.shape[1] * m_ref.shape[2]
                            * pl.num_programs(0) * pl.num_programs(1))
        tot1 = sums_ref[0, :]
        tot0 = sums_ref[1, :]
        ch_ref[0, :] = tot1 / n_ch
        unch_ref[0, :] = (tot0 - tot1) / (n_tot - n_ch)


def kernel(x, mask):
    B, C, H, W = x.shape
    grid = (B, H // _HT)
    unch, ch = pl.pallas_call(
        _pool_body,
        grid=grid,
        in_specs=[
            pl.BlockSpec((1, C, _HT, W), lambda b, h: (b, 0, h, 0)),
            pl.BlockSpec((1, _HT, W), lambda b, h: (b, h, 0)),
        ],
        out_specs=[
            pl.BlockSpec((1, C), lambda b, h: (0, 0)),
            pl.BlockSpec((1, C), lambda b, h: (0, 0)),
        ],
        out_shape=[
            jax.ShapeDtypeStruct((1, C), jnp.float32),
            jax.ShapeDtypeStruct((1, C), jnp.float32),
        ],
        compiler_params=pltpu.CompilerParams(
            vmem_limit_bytes=100 * 1024 * 1024),
        scratch_shapes=[
            pltpu.VMEM((2, C), jnp.float32),
            pltpu.SMEM((1, 1), jnp.float32),
        ],
    )(x, mask)
    return (unch.reshape(C), ch.reshape(C))


# speedup vs baseline: 1.0191x; 1.0191x over previous
import jax
import jax.numpy as jnp
from jax.experimental import pallas as pl
from jax.experimental.pallas import tpu as pltpu

_HT = 64


def _pool_body(x_ref, m_ref, unch_ref, ch_ref, sums_ref, cnt_ref):
    b = pl.program_id(0)
    h = pl.program_id(1)

    @pl.when((b == 0) & (h == 0))
    def _init():
        sums_ref[...] = jnp.zeros_like(sums_ref)
        cnt_ref[0, 0] = jnp.float32(0.0)

    xb = x_ref[0]
    mb = m_ref[0].astype(jnp.float32)
    s1 = jnp.sum(xb * mb[None, :, :], axis=(1, 2))
    s0 = jnp.sum(xb, axis=(1, 2))
    sums_ref[...] += jnp.stack([s1, s0])
    cnt_ref[0, 0] += jnp.sum(mb)

    @pl.when((b == pl.num_programs(0) - 1) & (h == pl.num_programs(1) - 1))
    def _finish():
        n_ch = cnt_ref[0, 0]
        n_tot = jnp.float32(m_ref.shape[1] * m_ref.shape[2]
                            * pl.num_programs(0) * pl.num_programs(1))
        tot1 = sums_ref[0, :]
        tot0 = sums_ref[1, :]
        ch_ref[0, :] = tot1 / n_ch
        unch_ref[0, :] = (tot0 - tot1) / (n_tot - n_ch)


def kernel(x, mask):
    B, C, H, W = x.shape
    grid = (B, H // _HT)
    unch, ch = pl.pallas_call(
        _pool_body,
        grid=grid,
        in_specs=[
            pl.BlockSpec((1, C, _HT, W), lambda b, h: (b, 0, h, 0)),
            pl.BlockSpec((1, _HT, W), lambda b, h: (b, h, 0)),
        ],
        out_specs=[
            pl.BlockSpec((1, C), lambda b, h: (0, 0)),
            pl.BlockSpec((1, C), lambda b, h: (0, 0)),
        ],
        out_shape=[
            jax.ShapeDtypeStruct((1, C), jnp.float32),
            jax.ShapeDtypeStruct((1, C), jnp.float32),
        ],
        scratch_shapes=[
            pltpu.VMEM((2, C), jnp.float32),
            pltpu.SMEM((1, 1), jnp.float32),
        ],
    )(x, mask)
    return (unch.reshape(C), ch.reshape(C))
